# Initial kernel scaffold; baseline (speedup 1.0000x reference)
import jax
import jax.numpy as jnp
from jax.experimental import pallas as pl


def kernel(y, y_est):
    raise NotImplementedError("write your pallas kernel here")


# speedup vs baseline: 16.7388x; 16.7388x over previous
import jax
import jax.numpy as jnp
from jax.experimental import pallas as pl
from jax.experimental.pallas import tpu as pltpu

_BINS = 100
_LANES = 128
_BR = 1024


def _minmax_body(y_ref, e_ref, min_ref, max_ref):
    @pl.when(pl.program_id(1) == 0)
    def _init():
        min_ref[...] = jnp.full(min_ref.shape, jnp.inf, jnp.float32)
        max_ref[...] = jnp.full(max_ref.shape, -jnp.inf, jnp.float32)

    lane = jax.lax.broadcasted_iota(jnp.int32, min_ref.shape, 2)
    ymin = jnp.min(y_ref[...])
    emin = jnp.min(e_ref[...])
    ymax = jnp.max(y_ref[...])
    emax = jnp.max(e_ref[...])
    minv = jnp.where(lane == 0, ymin, jnp.where(lane == 1, emin, jnp.inf))
    maxv = jnp.where(lane == 0, ymax, jnp.where(lane == 1, emax, -jnp.inf))
    min_ref[...] = jnp.minimum(min_ref[...], minv)
    max_ref[...] = jnp.maximum(max_ref[...], maxv)


def _hist_body(s_ref, y_ref, e_ref, hy_ref, he_ref, hy2, he2, *, g2):
    j = pl.program_id(1)

    @pl.when(j == 0)
    def _init():
        hy2[...] = jnp.zeros_like(hy2)
        he2[...] = jnp.zeros_like(he2)

    ymn = s_ref[0, 0]
    ysc = s_ref[0, 1]
    emn = s_ref[0, 2]
    esc = s_ref[0, 3]

    yidx = jnp.clip(jnp.floor((y_ref[...] - ymn) * ysc), 0.0, float(_BINS - 1))
    eidx = jnp.clip(jnp.floor((e_ref[...] - emn) * esc), 0.0, float(_BINS - 1))

    def body(b, carry):
        bf = b.astype(jnp.float32)
        hy2[pl.ds(b, 1), :] = hy2[pl.ds(b, 1), :] + jnp.sum(
            (yidx == bf).astype(jnp.float32), axis=0, keepdims=True)
        he2[pl.ds(b, 1), :] = he2[pl.ds(b, 1), :] + jnp.sum(
            (eidx == bf).astype(jnp.float32), axis=0, keepdims=True)
        return carry

    jax.lax.fori_loop(0, _BINS, body, 0)

    @pl.when(j == g2 - 1)
    def _flush():
        hy_ref[0] = hy2[...]
        he_ref[0] = he2[...]


def _combine_body(hy_ref, he_ref, loss_ref):
    d = jnp.sum(hy_ref[...] - he_ref[...], axis=0)
    row = jnp.sum(d, axis=1, keepdims=True)
    loss_ref[...] = jnp.sum(row * row).reshape(1, 1)


def kernel(y, y_est):
    total = y.size
    rows = total // _LANES
    y2 = y.reshape(rows, _LANES)
    e2 = y_est.reshape(rows, _LANES)

    ncore = 2
    rows_per_core = rows // ncore
    br = min(_BR, rows_per_core)
    g2 = rows_per_core // br

    data_spec = pl.BlockSpec((br, _LANES), lambda i, j: (i * g2 + j, 0))

    mins, maxs = pl.pallas_call(
        _minmax_body,
        grid=(ncore, g2),
        in_specs=[data_spec, data_spec],
        out_specs=[
            pl.BlockSpec((1, 8, _LANES), lambda i, j: (i, 0, 0)),
            pl.BlockSpec((1, 8, _LANES), lambda i, j: (i, 0, 0)),
        ],
        out_shape=[
            jax.ShapeDtypeStruct((ncore, 8, _LANES), jnp.float32),
            jax.ShapeDtypeStruct((ncore, 8, _LANES), jnp.float32),
        ],
        compiler_params=pltpu.CompilerParams(
            dimension_semantics=("parallel", "arbitrary")),
    )(y2, e2)

    ymn = jnp.min(mins[..., 0])
    emn = jnp.min(mins[..., 1])
    ymx = jnp.max(maxs[..., 0])
    emx = jnp.max(maxs[..., 1])
    one = jnp.float32(1.0)
    yrng = jnp.where(ymx > ymn, ymx - ymn, one)
    erng = jnp.where(emx > emn, emx - emn, one)
    params = jnp.zeros((1, _LANES), jnp.float32)
    params = params.at[0, 0].set(ymn).at[0, 1].set(_BINS / yrng)
    params = params.at[0, 2].set(emn).at[0, 3].set(_BINS / erng)

    import functools
    hy, he = pl.pallas_call(
        functools.partial(_hist_body, g2=g2),
        grid=(ncore, g2),
        in_specs=[
            pl.BlockSpec((1, _LANES), lambda i, j: (0, 0)),
            data_spec,
            data_spec,
        ],
        out_specs=[
            pl.BlockSpec((1, _LANES, _LANES), lambda i, j: (i, 0, 0)),
            pl.BlockSpec((1, _LANES, _LANES), lambda i, j: (i, 0, 0)),
        ],
        out_shape=[
            jax.ShapeDtypeStruct((ncore, _LANES, _LANES), jnp.float32),
            jax.ShapeDtypeStruct((ncore, _LANES, _LANES), jnp.float32),
        ],
        scratch_shapes=[
            pltpu.VMEM((_LANES, _LANES), jnp.float32),
            pltpu.VMEM((_LANES, _LANES), jnp.float32),
        ],
        compiler_params=pltpu.CompilerParams(
            dimension_semantics=("parallel", "arbitrary")),
    )(params, y2, e2)

    loss = pl.pallas_call(
        _combine_body,
        out_shape=jax.ShapeDtypeStruct((1, 1), jnp.float32),
    )(hy, he)

    return loss[0, 0]


# speedup vs baseline: 27.2238x; 1.6264x over previous
import dataclasses
import functools

import jax
import jax.numpy as jnp
from jax import lax
from jax.experimental import pallas as pl
from jax.experimental.pallas import tpu as pltpu
from jax.experimental.pallas import tpu_sc as plsc

_BINS = 100
_LANES = 128
_SC_L = 16
_CHUNK = 8192
_UNROLL = 4


def _minmax_body(y_ref, e_ref, min_ref, max_ref):
    @pl.when(pl.program_id(1) == 0)
    def _init():
        min_ref[...] = jnp.full(min_ref.shape, jnp.inf, jnp.float32)
        max_ref[...] = jnp.full(max_ref.shape, -jnp.inf, jnp.float32)

    lane = jax.lax.broadcasted_iota(jnp.int32, min_ref.shape, 2)
    ymin = jnp.min(y_ref[...])
    emin = jnp.min(e_ref[...])
    ymax = jnp.max(y_ref[...])
    emax = jnp.max(e_ref[...])
    minv = jnp.where(lane == 0, ymin, jnp.where(lane == 1, emin, jnp.inf))
    maxv = jnp.where(lane == 0, ymax, jnp.where(lane == 1, emax, -jnp.inf))
    min_ref[...] = jnp.minimum(min_ref[...], minv)
    max_ref[...] = jnp.maximum(max_ref[...], maxv)


def _tc_minmax(y2, e2):
    rows = y2.shape[0]
    ncore = 2
    rows_per_core = rows // ncore
    br = min(1024, rows_per_core)
    g2 = rows_per_core // br
    data_spec = pl.BlockSpec((br, _LANES), lambda i, j: (i * g2 + j, 0))
    return pl.pallas_call(
        _minmax_body,
        grid=(ncore, g2),
        in_specs=[data_spec, data_spec],
        out_specs=[
            pl.BlockSpec((1, 8, _LANES), lambda i, j: (i, 0, 0)),
            pl.BlockSpec((1, 8, _LANES), lambda i, j: (i, 0, 0)),
        ],
        out_shape=[
            jax.ShapeDtypeStruct((ncore, 8, _LANES), jnp.float32),
            jax.ShapeDtypeStruct((ncore, 8, _LANES), jnp.float32),
        ],
        compiler_params=pltpu.CompilerParams(
            dimension_semantics=("parallel", "arbitrary")),
    )(y2, e2)


def _sc_hist(yb, eb, params):
    nblk = yb.shape[0]
    mesh = plsc.VectorSubcoreMesh(core_axis_name="c", subcore_axis_name="s")
    acc_words = _SC_L * _LANES

    cp = pltpu.CompilerParams()
    if "needs_layout_passes" in pltpu.CompilerParams.__dataclass_fields__:
        cp = dataclasses.replace(cp, needs_layout_passes=False)

    @functools.partial(
        pl.kernel,
        mesh=mesh,
        compiler_params=cp,
        out_type=[
            jax.ShapeDtypeStruct((32, acc_words), jnp.float32),
            jax.ShapeDtypeStruct((32, acc_words), jnp.float32),
        ],
        scratch_types=[
            pltpu.VMEM((64,), jnp.float32),
            pltpu.VMEM((acc_words,), jnp.float32),
            pltpu.VMEM((acc_words,), jnp.float32),
        ],
    )
    def hist_kernel(y_hbm, e_hbm, p_hbm, oy_hbm, oe_hbm, p_v, hy_v, he_v):
        wid = lax.axis_index("s") * 2 + lax.axis_index("c")
        pltpu.sync_copy(p_hbm, p_v)
        ymn = p_v[pl.ds(0, _SC_L)]
        ysc = p_v[pl.ds(_SC_L, _SC_L)]
        emn = p_v[pl.ds(2 * _SC_L, _SC_L)]
        esc = p_v[pl.ds(3 * _SC_L, _SC_L)]
        zero16 = jnp.zeros((_SC_L,), jnp.float32)
        one16 = jnp.full((_SC_L,), 1.0, jnp.float32)
        i0 = jnp.zeros((_SC_L,), jnp.int32)
        i99 = jnp.full((_SC_L,), _BINS - 1, jnp.int32)
        loff = lax.iota(jnp.int32, _SC_L) * _LANES

        @pl.loop(0, acc_words, step=_SC_L)
        def _zero(i):
            hy_v[pl.ds(i, _SC_L)] = zero16
            he_v[pl.ds(i, _SC_L)] = zero16

        def body(y_blk, e_blk):
            yrow = y_blk.at[0]
            erow = e_blk.at[0]

            @pl.loop(0, _CHUNK, step=_SC_L * _UNROLL)
            def _(c):
                for u in range(_UNROLL):
                    off = c + u * _SC_L
                    x = yrow[pl.ds(off, _SC_L)]
                    xi = ((x - ymn) * ysc).astype(jnp.int32)
                    xi = jnp.maximum(jnp.minimum(xi, i99), i0) + loff
                    plsc.addupdate_scatter(hy_v, [xi], one16)
                    z = erow[pl.ds(off, _SC_L)]
                    zi = ((z - emn) * esc).astype(jnp.int32)
                    zi = jnp.maximum(jnp.minimum(zi, i99), i0) + loff
                    plsc.addupdate_scatter(he_v, [zi], one16)

        pltpu.emit_pipeline(
            body,
            grid=(nblk,),
            in_specs=[
                pl.BlockSpec((1, _CHUNK), lambda i: (i, 0)),
                pl.BlockSpec((1, _CHUNK), lambda i: (i, 0)),
            ],
            out_specs=[],
            core_axis_name=("c", "s"),
            dimension_semantics=(pltpu.PARALLEL,),
        )(y_hbm, e_hbm)

        pltpu.sync_copy(hy_v, oy_hbm.at[wid])
        pltpu.sync_copy(he_v, oe_hbm.at[wid])

    return hist_kernel(yb, eb, params)


def _combine_body(hy_ref, he_ref, loss_ref):
    d = jnp.sum(hy_ref[...] - he_ref[...], axis=0, keepdims=True)
    loss_ref[...] = jnp.sum(d * d).reshape(1, 1)


def kernel(y, y_est):
    total = y.size
    rows = total // _LANES
    y2 = y.reshape(rows, _LANES)
    e2 = y_est.reshape(rows, _LANES)

    mins, maxs = _tc_minmax(y2, e2)

    ymn = jnp.min(mins[..., 0])
    emn = jnp.min(mins[..., 1])
    ymx = jnp.max(maxs[..., 0])
    emx = jnp.max(maxs[..., 1])
    one = jnp.float32(1.0)
    yrng = jnp.where(ymx > ymn, ymx - ymn, one)
    erng = jnp.where(emx > emn, emx - emn, one)
    ysc = _BINS / yrng
    esc = _BINS / erng
    params = jnp.concatenate([
        jnp.full((_SC_L,), v, jnp.float32) for v in (ymn, ysc, emn, esc)
    ])

    nblk = total // _CHUNK
    hy, he = _sc_hist(y.reshape(nblk, _CHUNK), y_est.reshape(nblk, _CHUNK),
                      params)

    loss = pl.pallas_call(
        _combine_body,
        out_shape=jax.ShapeDtypeStruct((1, 1), jnp.float32),
    )(hy.reshape(-1, _LANES), he.reshape(-1, _LANES))

    return loss[0, 0]


# speedup vs baseline: 41.1143x; 1.5102x over previous
import dataclasses
import functools

import jax
import jax.numpy as jnp
from jax import lax
from jax.experimental import pallas as pl
from jax.experimental.pallas import tpu as pltpu
from jax.experimental.pallas import tpu_sc as plsc

_BINS = 100
_LANES = 128
_SC_L = 16
_CHUNK = 16384
_UNROLL = 8
_MM_BC = 8192


def _minmax_body(y_ref, e_ref, min_ref, max_ref):
    @pl.when(pl.program_id(1) == 0)
    def _init():
        min_ref[...] = jnp.full(min_ref.shape, jnp.inf, jnp.float32)
        max_ref[...] = jnp.full(max_ref.shape, -jnp.inf, jnp.float32)

    lane = jax.lax.broadcasted_iota(jnp.int32, min_ref.shape, 2)
    ymin = jnp.min(y_ref[...])
    emin = jnp.min(e_ref[...])
    ymax = jnp.max(y_ref[...])
    emax = jnp.max(e_ref[...])
    minv = jnp.where(lane == 0, ymin, jnp.where(lane == 1, emin, jnp.inf))
    maxv = jnp.where(lane == 0, ymax, jnp.where(lane == 1, emax, -jnp.inf))
    min_ref[...] = jnp.minimum(min_ref[...], minv)
    max_ref[...] = jnp.maximum(max_ref[...], maxv)


def _tc_minmax(y, e):
    b, n = y.shape
    ncore = 2
    g2 = n // (_MM_BC * ncore)
    data_spec = pl.BlockSpec((b, _MM_BC), lambda i, j: (0, i * g2 + j))
    return pl.pallas_call(
        _minmax_body,
        grid=(ncore, g2),
        in_specs=[data_spec, data_spec],
        out_specs=[
            pl.BlockSpec((1, 8, _LANES), lambda i, j: (i, 0, 0)),
            pl.BlockSpec((1, 8, _LANES), lambda i, j: (i, 0, 0)),
        ],
        out_shape=[
            jax.ShapeDtypeStruct((ncore, 8, _LANES), jnp.float32),
            jax.ShapeDtypeStruct((ncore, 8, _LANES), jnp.float32),
        ],
        compiler_params=pltpu.CompilerParams(
            dimension_semantics=("parallel", "arbitrary")),
    )(y, e)


def _sc_hist(yb, eb, params):
    b, n = yb.shape
    ncol = n // _CHUNK
    mesh = plsc.VectorSubcoreMesh(core_axis_name="c", subcore_axis_name="s")
    acc_words = _SC_L * _LANES

    cp = pltpu.CompilerParams()
    if "needs_layout_passes" in pltpu.CompilerParams.__dataclass_fields__:
        cp = dataclasses.replace(cp, needs_layout_passes=False)

    @functools.partial(
        pl.kernel,
        mesh=mesh,
        compiler_params=cp,
        out_type=[
            jax.ShapeDtypeStruct((32, acc_words), jnp.float32),
            jax.ShapeDtypeStruct((32, acc_words), jnp.float32),
        ],
        scratch_types=[
            pltpu.VMEM((64,), jnp.float32),
            pltpu.VMEM((acc_words,), jnp.float32),
            pltpu.VMEM((acc_words,), jnp.float32),
        ],
    )
    def hist_kernel(y_hbm, e_hbm, p_hbm, oy_hbm, oe_hbm, p_v, hy_v, he_v):
        wid = lax.axis_index("s") * 2 + lax.axis_index("c")
        pltpu.sync_copy(p_hbm, p_v)
        ymn = p_v[pl.ds(0, _SC_L)]
        ysc = p_v[pl.ds(_SC_L, _SC_L)]
        emn = p_v[pl.ds(2 * _SC_L, _SC_L)]
        esc = p_v[pl.ds(3 * _SC_L, _SC_L)]
        zero16 = jnp.zeros((_SC_L,), jnp.float32)
        one16 = jnp.full((_SC_L,), 1.0, jnp.float32)
        i0 = jnp.zeros((_SC_L,), jnp.int32)
        i99 = jnp.full((_SC_L,), _BINS - 1, jnp.int32)
        loff = lax.iota(jnp.int32, _SC_L) * _LANES

        @pl.loop(0, acc_words, step=_SC_L)
        def _zero(i):
            hy_v[pl.ds(i, _SC_L)] = zero16
            he_v[pl.ds(i, _SC_L)] = zero16

        def body(y_blk, e_blk):
            yrow = y_blk.at[0]
            erow = e_blk.at[0]

            @pl.loop(0, _CHUNK, step=_SC_L * _UNROLL)
            def _(c):
                for u in range(_UNROLL):
                    off = c + u * _SC_L
                    x = yrow[pl.ds(off, _SC_L)]
                    xi = ((x - ymn) * ysc).astype(jnp.int32)
                    xi = jnp.maximum(jnp.minimum(xi, i99), i0) + loff
                    plsc.addupdate_scatter(hy_v, [xi], one16)
                    z = erow[pl.ds(off, _SC_L)]
                    zi = ((z - emn) * esc).astype(jnp.int32)
                    zi = jnp.maximum(jnp.minimum(zi, i99), i0) + loff
                    plsc.addupdate_scatter(he_v, [zi], one16)

        pltpu.emit_pipeline(
            body,
            grid=(b, ncol),
            in_specs=[
                pl.BlockSpec((1, _CHUNK), lambda i, j: (i, j)),
                pl.BlockSpec((1, _CHUNK), lambda i, j: (i, j)),
            ],
            out_specs=[],
            core_axis_name=("c", "s"),
            dimension_semantics=(pltpu.PARALLEL, pltpu.PARALLEL),
        )(y_hbm, e_hbm)

        pltpu.sync_copy(hy_v, oy_hbm.at[wid])
        pltpu.sync_copy(he_v, oe_hbm.at[wid])

    return hist_kernel(yb, eb, params)


def _combine_body(hy_ref, he_ref, loss_ref):
    d = jnp.sum(hy_ref[...] - he_ref[...], axis=0, keepdims=True)
    loss_ref[...] = jnp.sum(d * d).reshape(1, 1)


def kernel(y, y_est):
    mins, maxs = _tc_minmax(y, y_est)

    ymn = jnp.min(mins[..., 0])
    emn = jnp.min(mins[..., 1])
    ymx = jnp.max(maxs[..., 0])
    emx = jnp.max(maxs[..., 1])
    one = jnp.float32(1.0)
    yrng = jnp.where(ymx > ymn, ymx - ymn, one)
    erng = jnp.where(emx > emn, emx - emn, one)
    ysc = _BINS / yrng
    esc = _BINS / erng
    params = jnp.concatenate([
        jnp.full((_SC_L,), v, jnp.float32) for v in (ymn, ysc, emn, esc)
    ])

    hy, he = _sc_hist(y, y_est, params)

    loss = pl.pallas_call(
        _combine_body,
        out_shape=jax.ShapeDtypeStruct((1, 1), jnp.float32),
    )(hy.reshape(-1, _LANES), he.reshape(-1, _LANES))

    return loss[0, 0]


# speedup vs baseline: 175.5388x; 4.2695x over previous
import dataclasses
import functools

import jax
import jax.numpy as jnp
from jax import lax
from jax.experimental import pallas as pl
from jax.experimental.pallas import tpu as pltpu
from jax.experimental.pallas import tpu_sc as plsc

_BINS = 100
_LANES = 128
_SC_L = 16
_CHUNK = 16384
_UNROLL = 8
_MM_BC = 8192


def _minmax_body(y_ref, e_ref, min_ref, max_ref):
    @pl.when(pl.program_id(1) == 0)
    def _init():
        min_ref[...] = jnp.full(min_ref.shape, jnp.inf, jnp.float32)
        max_ref[...] = jnp.full(max_ref.shape, -jnp.inf, jnp.float32)

    lane = jax.lax.broadcasted_iota(jnp.int32, min_ref.shape, 2)
    ymin = jnp.min(y_ref[...])
    emin = jnp.min(e_ref[...])
    ymax = jnp.max(y_ref[...])
    emax = jnp.max(e_ref[...])
    minv = jnp.where(lane == 0, ymin, jnp.where(lane == 1, emin, jnp.inf))
    maxv = jnp.where(lane == 0, ymax, jnp.where(lane == 1, emax, -jnp.inf))
    min_ref[...] = jnp.minimum(min_ref[...], minv)
    max_ref[...] = jnp.maximum(max_ref[...], maxv)


def _tc_minmax(y, e):
    b, n = y.shape
    ncore = 2
    g2 = n // (_MM_BC * ncore)
    data_spec = pl.BlockSpec((b, _MM_BC), lambda i, j: (0, i * g2 + j))
    return pl.pallas_call(
        _minmax_body,
        grid=(ncore, g2),
        in_specs=[data_spec, data_spec],
        out_specs=[
            pl.BlockSpec((1, 8, _LANES), lambda i, j: (i, 0, 0)),
            pl.BlockSpec((1, 8, _LANES), lambda i, j: (i, 0, 0)),
        ],
        out_shape=[
            jax.ShapeDtypeStruct((ncore, 8, _LANES), jnp.float32),
            jax.ShapeDtypeStruct((ncore, 8, _LANES), jnp.float32),
        ],
        compiler_params=pltpu.CompilerParams(
            dimension_semantics=("parallel", "arbitrary")),
    )(y, e)


def _sc_hist(yb, eb, params):
    b, n = yb.shape
    ncol = n // _CHUNK
    mesh = plsc.VectorSubcoreMesh(core_axis_name="c", subcore_axis_name="s")
    acc_words = _SC_L * _LANES

    cp = pltpu.CompilerParams()
    if "needs_layout_passes" in pltpu.CompilerParams.__dataclass_fields__:
        cp = dataclasses.replace(cp, needs_layout_passes=False)

    @functools.partial(
        pl.kernel,
        mesh=mesh,
        compiler_params=cp,
        out_type=[
            jax.ShapeDtypeStruct((32, acc_words), jnp.float32),
            jax.ShapeDtypeStruct((32, acc_words), jnp.float32),
        ],
        scratch_types=[
            pltpu.VMEM((64,), jnp.float32),
            pltpu.VMEM((acc_words,), jnp.float32),
            pltpu.VMEM((acc_words,), jnp.float32),
        ],
    )
    def hist_kernel(y_hbm, e_hbm, p_hbm, oy_hbm, oe_hbm, p_v, hy_v, he_v):
        wid = lax.axis_index("s") * 2 + lax.axis_index("c")
        pltpu.sync_copy(p_hbm, p_v)
        ymn = p_v[pl.ds(0, _SC_L)]
        ysc = p_v[pl.ds(_SC_L, _SC_L)]
        emn = p_v[pl.ds(2 * _SC_L, _SC_L)]
        esc = p_v[pl.ds(3 * _SC_L, _SC_L)]
        zero16 = jnp.zeros((_SC_L,), jnp.float32)
        one16 = jnp.full((_SC_L,), 1.0, jnp.float32)
        i0 = jnp.zeros((_SC_L,), jnp.int32)
        i99 = jnp.full((_SC_L,), _BINS - 1, jnp.int32)
        loff = lax.iota(jnp.int32, _SC_L) * _LANES

        @pl.loop(0, acc_words, step=_SC_L)
        def _zero(i):
            hy_v[pl.ds(i, _SC_L)] = zero16
            he_v[pl.ds(i, _SC_L)] = zero16

        def body(y_blk, e_blk):
            yrow = y_blk.at[0]
            erow = e_blk.at[0]

            @plsc.parallel_loop(0, _CHUNK, step=_SC_L, unroll=_UNROLL)
            def _(c):
                x = yrow[pl.ds(c, _SC_L)]
                xi = ((x - ymn) * ysc).astype(jnp.int32)
                xi = jnp.maximum(jnp.minimum(xi, i99), i0) + loff
                plsc.addupdate_scatter(hy_v, [xi], one16)
                z = erow[pl.ds(c, _SC_L)]
                zi = ((z - emn) * esc).astype(jnp.int32)
                zi = jnp.maximum(jnp.minimum(zi, i99), i0) + loff
                plsc.addupdate_scatter(he_v, [zi], one16)

        pltpu.emit_pipeline(
            body,
            grid=(b, ncol),
            in_specs=[
                pl.BlockSpec((1, _CHUNK), lambda i, j: (i, j)),
                pl.BlockSpec((1, _CHUNK), lambda i, j: (i, j)),
            ],
            out_specs=[],
            core_axis_name=("c", "s"),
            dimension_semantics=(pltpu.PARALLEL, pltpu.PARALLEL),
        )(y_hbm, e_hbm)

        pltpu.sync_copy(hy_v, oy_hbm.at[wid])
        pltpu.sync_copy(he_v, oe_hbm.at[wid])

    return hist_kernel(yb, eb, params)


def _combine_body(hy_ref, he_ref, loss_ref):
    d = jnp.sum(hy_ref[...] - he_ref[...], axis=0, keepdims=True)
    loss_ref[...] = jnp.sum(d * d).reshape(1, 1)


def kernel(y, y_est):
    mins, maxs = _tc_minmax(y, y_est)

    ymn = jnp.min(mins[..., 0])
    emn = jnp.min(mins[..., 1])
    ymx = jnp.max(maxs[..., 0])
    emx = jnp.max(maxs[..., 1])
    one = jnp.float32(1.0)
    yrng = jnp.where(ymx > ymn, ymx - ymn, one)
    erng = jnp.where(emx > emn, emx - emn, one)
    ysc = _BINS / yrng
    esc = _BINS / erng
    params = jnp.concatenate([
        jnp.full((_SC_L,), v, jnp.float32) for v in (ymn, ysc, emn, esc)
    ])

    hy, he = _sc_hist(y, y_est, params)

    loss = pl.pallas_call(
        _combine_body,
        out_shape=jax.ShapeDtypeStruct((1, 1), jnp.float32),
    )(hy.reshape(-1, _LANES), he.reshape(-1, _LANES))

    return loss[0, 0]


# speedup vs baseline: 209.5018x; 1.1935x over previous
import dataclasses
import functools

import jax
import jax.numpy as jnp
from jax import lax
from jax.experimental import pallas as pl
from jax.experimental.pallas import tpu as pltpu
from jax.experimental.pallas import tpu_sc as plsc

_BINS = 100
_LANES = 128
_SC_L = 16
_CHUNK = 16384
_UNROLL = 16
_MM_BC = 16384


def _minmax_body(y_ref, e_ref, min_ref, max_ref):
    @pl.when(pl.program_id(1) == 0)
    def _init():
        min_ref[...] = jnp.full(min_ref.shape, jnp.inf, jnp.float32)
        max_ref[...] = jnp.full(max_ref.shape, -jnp.inf, jnp.float32)

    lane = jax.lax.broadcasted_iota(jnp.int32, min_ref.shape, 2)
    ymin = jnp.min(y_ref[...])
    emin = jnp.min(e_ref[...])
    ymax = jnp.max(y_ref[...])
    emax = jnp.max(e_ref[...])
    minv = jnp.where(lane == 0, ymin, jnp.where(lane == 1, emin, jnp.inf))
    maxv = jnp.where(lane == 0, ymax, jnp.where(lane == 1, emax, -jnp.inf))
    min_ref[...] = jnp.minimum(min_ref[...], minv)
    max_ref[...] = jnp.maximum(max_ref[...], maxv)


def _tc_minmax(y, e):
    b, n = y.shape
    ncore = 2
    g2 = n // (_MM_BC * ncore)
    data_spec = pl.BlockSpec((b, _MM_BC), lambda i, j: (0, i * g2 + j))
    return pl.pallas_call(
        _minmax_body,
        grid=(ncore, g2),
        in_specs=[data_spec, data_spec],
        out_specs=[
            pl.BlockSpec((1, 8, _LANES), lambda i, j: (i, 0, 0)),
            pl.BlockSpec((1, 8, _LANES), lambda i, j: (i, 0, 0)),
        ],
        out_shape=[
            jax.ShapeDtypeStruct((ncore, 8, _LANES), jnp.float32),
            jax.ShapeDtypeStruct((ncore, 8, _LANES), jnp.float32),
        ],
        compiler_params=pltpu.CompilerParams(
            dimension_semantics=("parallel", "arbitrary")),
    )(y, e)


def _sc_hist(yb, eb, params):
    b, n = yb.shape
    ncol = n // _CHUNK
    mesh = plsc.VectorSubcoreMesh(core_axis_name="c", subcore_axis_name="s")
    acc_words = _SC_L * _LANES

    cp = pltpu.CompilerParams()
    if "needs_layout_passes" in pltpu.CompilerParams.__dataclass_fields__:
        cp = dataclasses.replace(cp, needs_layout_passes=False)

    @functools.partial(
        pl.kernel,
        mesh=mesh,
        compiler_params=cp,
        out_type=[
            jax.ShapeDtypeStruct((32, acc_words), jnp.float32),
            jax.ShapeDtypeStruct((32, acc_words), jnp.float32),
        ],
        scratch_types=[
            pltpu.VMEM((64,), jnp.float32),
            pltpu.VMEM((acc_words,), jnp.float32),
            pltpu.VMEM((acc_words,), jnp.float32),
        ],
    )
    def hist_kernel(y_hbm, e_hbm, p_hbm, oy_hbm, oe_hbm, p_v, hy_v, he_v):
        wid = lax.axis_index("s") * 2 + lax.axis_index("c")
        pltpu.sync_copy(p_hbm, p_v)
        ymn = p_v[pl.ds(0, _SC_L)]
        ysc = p_v[pl.ds(_SC_L, _SC_L)]
        emn = p_v[pl.ds(2 * _SC_L, _SC_L)]
        esc = p_v[pl.ds(3 * _SC_L, _SC_L)]
        zero16 = jnp.zeros((_SC_L,), jnp.float32)
        one16 = jnp.full((_SC_L,), 1.0, jnp.float32)
        i99 = jnp.full((_SC_L,), _BINS - 1, jnp.int32)
        loff = lax.iota(jnp.int32, _SC_L) * _LANES

        @pl.loop(0, acc_words, step=_SC_L)
        def _zero(i):
            hy_v[pl.ds(i, _SC_L)] = zero16
            he_v[pl.ds(i, _SC_L)] = zero16

        def body(y_blk, e_blk):
            yrow = y_blk.at[0]
            erow = e_blk.at[0]

            @plsc.parallel_loop(0, _CHUNK, step=_SC_L, unroll=_UNROLL)
            def _(c):
                x = yrow[pl.ds(c, _SC_L)]
                xi = ((x - ymn) * ysc).astype(jnp.int32)
                xi = jnp.minimum(xi, i99) + loff
                plsc.addupdate_scatter(hy_v, [xi], one16)
                z = erow[pl.ds(c, _SC_L)]
                zi = ((z - emn) * esc).astype(jnp.int32)
                zi = jnp.minimum(zi, i99) + loff
                plsc.addupdate_scatter(he_v, [zi], one16)

        pltpu.emit_pipeline(
            body,
            grid=(b, ncol),
            in_specs=[
                pl.BlockSpec((1, _CHUNK), lambda i, j: (i, j)),
                pl.BlockSpec((1, _CHUNK), lambda i, j: (i, j)),
            ],
            out_specs=[],
            core_axis_name=("c", "s"),
            dimension_semantics=(pltpu.PARALLEL, pltpu.PARALLEL),
        )(y_hbm, e_hbm)

        pltpu.sync_copy(hy_v, oy_hbm.at[wid])
        pltpu.sync_copy(he_v, oe_hbm.at[wid])

    return hist_kernel(yb, eb, params)


def _combine_body(hy_ref, he_ref, loss_ref):
    d = jnp.sum(hy_ref[...] - he_ref[...], axis=0, keepdims=True)
    loss_ref[...] = jnp.sum(d * d).reshape(1, 1)


def kernel(y, y_est):
    mins, maxs = _tc_minmax(y, y_est)

    ymn = jnp.min(mins[..., 0])
    emn = jnp.min(mins[..., 1])
    ymx = jnp.max(maxs[..., 0])
    emx = jnp.max(maxs[..., 1])
    one = jnp.float32(1.0)
    yrng = jnp.where(ymx > ymn, ymx - ymn, one)
    erng = jnp.where(emx > emn, emx - emn, one)
    ysc = _BINS / yrng
    esc = _BINS / erng
    params = jnp.concatenate([
        jnp.full((_SC_L,), v, jnp.float32) for v in (ymn, ysc, emn, esc)
    ])

    hy, he = _sc_hist(y, y_est, params)

    loss = pl.pallas_call(
        _combine_body,
        out_shape=jax.ShapeDtypeStruct((1, 1), jnp.float32),
    )(hy.reshape(-1, _LANES), he.reshape(-1, _LANES))

    return loss[0, 0]


# speedup vs baseline: 219.7064x; 1.0487x over previous
import dataclasses
import functools

import jax
import jax.numpy as jnp
from jax import lax
from jax.experimental import pallas as pl
from jax.experimental.pallas import tpu as pltpu
from jax.experimental.pallas import tpu_sc as plsc

_BINS = 100
_LANES = 128
_SC_L = 16
_CHUNK = 16384
_UNROLL = 16
_MM_BC = 32768


def _minmax_body(y_ref, e_ref, min_ref, max_ref):
    @pl.when(pl.program_id(1) == 0)
    def _init():
        min_ref[...] = jnp.full(min_ref.shape, jnp.inf, jnp.float32)
        max_ref[...] = jnp.full(max_ref.shape, -jnp.inf, jnp.float32)

    lane = jax.lax.broadcasted_iota(jnp.int32, min_ref.shape, 2)
    ymin = jnp.min(y_ref[...])
    emin = jnp.min(e_ref[...])
    ymax = jnp.max(y_ref[...])
    emax = jnp.max(e_ref[...])
    minv = jnp.where(lane == 0, ymin, jnp.where(lane == 1, emin, jnp.inf))
    maxv = jnp.where(lane == 0, ymax, jnp.where(lane == 1, emax, -jnp.inf))
    min_ref[...] = jnp.minimum(min_ref[...], minv)
    max_ref[...] = jnp.maximum(max_ref[...], maxv)


def _tc_minmax(y, e):
    b, n = y.shape
    ncore = 2
    g2 = n // (_MM_BC * ncore)
    data_spec = pl.BlockSpec((b, _MM_BC), lambda i, j: (0, i * g2 + j))
    return pl.pallas_call(
        _minmax_body,
        grid=(ncore, g2),
        in_specs=[data_spec, data_spec],
        out_specs=[
            pl.BlockSpec((1, 8, _LANES), lambda i, j: (i, 0, 0)),
            pl.BlockSpec((1, 8, _LANES), lambda i, j: (i, 0, 0)),
        ],
        out_shape=[
            jax.ShapeDtypeStruct((ncore, 8, _LANES), jnp.float32),
            jax.ShapeDtypeStruct((ncore, 8, _LANES), jnp.float32),
        ],
        compiler_params=pltpu.CompilerParams(
            dimension_semantics=("parallel", "arbitrary")),
    )(y, e)


def _sc_hist(yb, eb, params):
    b, n = yb.shape
    ncol = n // _CHUNK
    mesh = plsc.VectorSubcoreMesh(core_axis_name="c", subcore_axis_name="s")
    acc_words = _SC_L * _LANES

    cp = pltpu.CompilerParams()
    if "needs_layout_passes" in pltpu.CompilerParams.__dataclass_fields__:
        cp = dataclasses.replace(cp, needs_layout_passes=False)

    @functools.partial(
        pl.kernel,
        mesh=mesh,
        compiler_params=cp,
        out_type=[
            jax.ShapeDtypeStruct((32 * _SC_L, _LANES), jnp.float32),
            jax.ShapeDtypeStruct((32 * _SC_L, _LANES), jnp.float32),
        ],
        scratch_types=[
            pltpu.VMEM((64,), jnp.float32),
            pltpu.VMEM((_SC_L, _LANES), jnp.float32),
            pltpu.VMEM((_SC_L, _LANES), jnp.float32),
        ],
    )
    def hist_kernel(y_hbm, e_hbm, p_hbm, oy_hbm, oe_hbm, p_v, hy_v, he_v):
        wid = lax.axis_index("s") * 2 + lax.axis_index("c")
        pltpu.sync_copy(p_hbm, p_v)
        yoff = p_v[pl.ds(0, _SC_L)]
        ysc = p_v[pl.ds(_SC_L, _SC_L)]
        eoff = p_v[pl.ds(2 * _SC_L, _SC_L)]
        esc = p_v[pl.ds(3 * _SC_L, _SC_L)]
        zero16 = jnp.zeros((_SC_L,), jnp.float32)
        one16 = jnp.full((_SC_L,), 1.0, jnp.float32)
        i99 = jnp.full((_SC_L,), _BINS - 1, jnp.int32)
        lane = lax.iota(jnp.int32, _SC_L)

        @pl.loop(0, _SC_L)
        def _zero(i):
            for j in range(0, _LANES, _SC_L):
                hy_v.at[i][pl.ds(j, _SC_L)] = zero16
                he_v.at[i][pl.ds(j, _SC_L)] = zero16

        def body(y_blk, e_blk):
            yrow = y_blk.at[0]
            erow = e_blk.at[0]

            @plsc.parallel_loop(0, _CHUNK, step=_SC_L, unroll=_UNROLL)
            def _(c):
                x = yrow[pl.ds(c, _SC_L)]
                xi = (x * ysc - yoff).astype(jnp.int32)
                xi = jnp.minimum(xi, i99)
                plsc.addupdate_scatter(hy_v, [lane, xi], one16)
                z = erow[pl.ds(c, _SC_L)]
                zi = (z * esc - eoff).astype(jnp.int32)
                zi = jnp.minimum(zi, i99)
                plsc.addupdate_scatter(he_v, [lane, zi], one16)

        pltpu.emit_pipeline(
            body,
            grid=(b, ncol),
            in_specs=[
                pl.BlockSpec((1, _CHUNK), lambda i, j: (i, j)),
                pl.BlockSpec((1, _CHUNK), lambda i, j: (i, j)),
            ],
            out_specs=[],
            core_axis_name=("c", "s"),
            dimension_semantics=(pltpu.PARALLEL, pltpu.PARALLEL),
        )(y_hbm, e_hbm)

        pltpu.sync_copy(hy_v, oy_hbm.at[pl.ds(wid * _SC_L, _SC_L)])
        pltpu.sync_copy(he_v, oe_hbm.at[pl.ds(wid * _SC_L, _SC_L)])

    return hist_kernel(yb, eb, params)


def _combine_body(hy_ref, he_ref, loss_ref):
    d = jnp.sum(hy_ref[...] - he_ref[...], axis=0, keepdims=True)
    loss_ref[...] = jnp.sum(d * d).reshape(1, 1)


def kernel(y, y_est):
    mins, maxs = _tc_minmax(y, y_est)

    ymn = jnp.min(mins[..., 0])
    emn = jnp.min(mins[..., 1])
    ymx = jnp.max(maxs[..., 0])
    emx = jnp.max(maxs[..., 1])
    one = jnp.float32(1.0)
    yrng = jnp.where(ymx > ymn, ymx - ymn, one)
    erng = jnp.where(emx > emn, emx - emn, one)
    ysc = _BINS / yrng
    esc = _BINS / erng
    yoff = ymn * ysc
    eoff = emn * esc
    params = jnp.concatenate([
        jnp.full((_SC_L,), v, jnp.float32) for v in (yoff, ysc, eoff, esc)
    ])

    hy, he = _sc_hist(y, y_est, params)

    loss = pl.pallas_call(
        _combine_body,
        out_shape=jax.ShapeDtypeStruct((1, 1), jnp.float32),
    )(hy, he)

    return loss[0, 0]


# speedup vs baseline: 249.5200x; 1.1357x over previous
import dataclasses
import functools

import jax
import jax.numpy as jnp
from jax import lax
from jax.experimental import pallas as pl
from jax.experimental.pallas import tpu as pltpu
from jax.experimental.pallas import tpu_sc as plsc

_BINS = 100
_LANES = 128
_SC_L = 16
_CHUNK = 16384
_UNROLL = 16
_MM_BC = 32768


def _minmax_body(y_ref, e_ref, min_ref, max_ref):
    @pl.when(pl.program_id(1) == 0)
    def _init():
        min_ref[...] = jnp.full(min_ref.shape, jnp.inf, jnp.float32)
        max_ref[...] = jnp.full(max_ref.shape, -jnp.inf, jnp.float32)

    lane = jax.lax.broadcasted_iota(jnp.int32, min_ref.shape, 2)
    ymin = jnp.min(y_ref[...])
    emin = jnp.min(e_ref[...])
    ymax = jnp.max(y_ref[...])
    emax = jnp.max(e_ref[...])
    minv = jnp.where(lane == 0, ymin, jnp.where(lane == 1, emin, jnp.inf))
    maxv = jnp.where(lane == 0, ymax, jnp.where(lane == 1, emax, -jnp.inf))
    min_ref[...] = jnp.minimum(min_ref[...], minv)
    max_ref[...] = jnp.maximum(max_ref[...], maxv)


def _tc_minmax(y, e):
    b, n = y.shape
    ncore = 2
    g2 = n // (_MM_BC * ncore)
    data_spec = pl.BlockSpec((b, _MM_BC), lambda i, j: (0, i * g2 + j))
    return pl.pallas_call(
        _minmax_body,
        grid=(ncore, g2),
        in_specs=[data_spec, data_spec],
        out_specs=[
            pl.BlockSpec((1, 8, _LANES), lambda i, j: (i, 0, 0)),
            pl.BlockSpec((1, 8, _LANES), lambda i, j: (i, 0, 0)),
        ],
        out_shape=[
            jax.ShapeDtypeStruct((ncore, 8, _LANES), jnp.float32),
            jax.ShapeDtypeStruct((ncore, 8, _LANES), jnp.float32),
        ],
        compiler_params=pltpu.CompilerParams(
            dimension_semantics=("parallel", "arbitrary")),
    )(y, e)


def _sc_hist(yb, eb, params):
    b, n = yb.shape
    ncol = n // _CHUNK
    mesh = plsc.VectorSubcoreMesh(core_axis_name="c", subcore_axis_name="s")
    acc_words = _SC_L * _LANES

    cp = pltpu.CompilerParams()
    if "needs_layout_passes" in pltpu.CompilerParams.__dataclass_fields__:
        cp = dataclasses.replace(cp, needs_layout_passes=False)

    @functools.partial(
        pl.kernel,
        mesh=mesh,
        compiler_params=cp,
        out_type=[
            jax.ShapeDtypeStruct((32, _LANES, _SC_L), jnp.float32),
            jax.ShapeDtypeStruct((32, _LANES, _SC_L), jnp.float32),
        ],
        scratch_types=[
            pltpu.VMEM((64,), jnp.float32),
            pltpu.VMEM((_LANES, _SC_L), jnp.float32),
            pltpu.VMEM((_LANES, _SC_L), jnp.float32),
        ],
    )
    def hist_kernel(y_hbm, e_hbm, p_hbm, oy_hbm, oe_hbm, p_v, hy_v, he_v):
        wid = lax.axis_index("s") * 2 + lax.axis_index("c")
        pltpu.sync_copy(p_hbm, p_v)
        yoff = p_v[pl.ds(0, _SC_L)]
        ysc = p_v[pl.ds(_SC_L, _SC_L)]
        eoff = p_v[pl.ds(2 * _SC_L, _SC_L)]
        esc = p_v[pl.ds(3 * _SC_L, _SC_L)]
        zero16 = jnp.zeros((_SC_L,), jnp.float32)
        one16 = jnp.full((_SC_L,), 1.0, jnp.float32)
        i99 = jnp.full((_SC_L,), _BINS - 1, jnp.int32)
        lane = lax.iota(jnp.int32, _SC_L)

        @pl.loop(0, _LANES)
        def _zero(i):
            hy_v.at[i][...] = zero16
            he_v.at[i][...] = zero16

        def body(y_blk, e_blk):
            yrow = y_blk.at[0]
            erow = e_blk.at[0]

            @plsc.parallel_loop(0, _CHUNK, step=_SC_L, unroll=_UNROLL)
            def _(c):
                x = yrow[pl.ds(c, _SC_L)]
                xi = (x * ysc - yoff).astype(jnp.int32)
                xi = jnp.minimum(xi, i99)
                plsc.addupdate_scatter(hy_v, [xi, lane], one16)
                z = erow[pl.ds(c, _SC_L)]
                zi = (z * esc - eoff).astype(jnp.int32)
                zi = jnp.minimum(zi, i99)
                plsc.addupdate_scatter(he_v, [zi, lane], one16)

        pltpu.emit_pipeline(
            body,
            grid=(b, ncol),
            in_specs=[
                pl.BlockSpec((1, _CHUNK), lambda i, j: (i, j)),
                pl.BlockSpec((1, _CHUNK), lambda i, j: (i, j)),
            ],
            out_specs=[],
            core_axis_name=("c", "s"),
            dimension_semantics=(pltpu.PARALLEL, pltpu.PARALLEL),
        )(y_hbm, e_hbm)

        pltpu.sync_copy(hy_v, oy_hbm.at[wid])
        pltpu.sync_copy(he_v, oe_hbm.at[wid])

    return hist_kernel(yb, eb, params)


def _combine_body(hy_ref, he_ref, loss_ref):
    d = hy_ref[...] - he_ref[...]
    s = jnp.sum(d, axis=0)
    s = jnp.sum(s, axis=1, keepdims=True)
    loss_ref[...] = jnp.sum(s * s).reshape(1, 1)


def kernel(y, y_est):
    mins, maxs = _tc_minmax(y, y_est)

    ymn = jnp.min(mins[..., 0])
    emn = jnp.min(mins[..., 1])
    ymx = jnp.max(maxs[..., 0])
    emx = jnp.max(maxs[..., 1])
    one = jnp.float32(1.0)
    yrng = jnp.where(ymx > ymn, ymx - ymn, one)
    erng = jnp.where(emx > emn, emx - emn, one)
    ysc = _BINS / yrng
    esc = _BINS / erng
    yoff = ymn * ysc
    eoff = emn * esc
    params = jnp.concatenate([
        jnp.full((_SC_L,), v, jnp.float32) for v in (yoff, ysc, eoff, esc)
    ])

    hy, he = _sc_hist(y, y_est, params)

    loss = pl.pallas_call(
        _combine_body,
        out_shape=jax.ShapeDtypeStruct((1, 1), jnp.float32),
    )(hy, he)

    return loss[0, 0]
